# Initial kernel scaffold; baseline (speedup 1.0000x reference)
import jax
import jax.numpy as jnp
from jax.experimental import pallas as pl


def kernel(x, edge_index, batch, W1, b1, W2, b2, lin_W, lin_b):
    raise NotImplementedError("write your pallas kernel here")


# speedup vs baseline: 14.4192x; 14.4192x over previous
import functools

import jax
import jax.numpy as jnp
from jax import lax
from jax.experimental import pallas as pl
from jax.experimental.pallas import tpu as pltpu
from jax.experimental.pallas import tpu_sc as plsc

N = 10000
E = 320000
D = 128
G = 256
NC = 2
NS = 16
NW = NC * NS
C = 128
CHUNKS_W = 79
E_PAD = NW * CHUNKS_W * C
EDGES_W = CHUNKS_W * C
N_ACC = NW * 320
ROWS_T = N_ACC // NS
RB = 400
NBLK = N // RB

def _worker_id():
    return lax.axis_index("s") * NC + lax.axis_index("c")


def _deg_body(dst_hbm, ones_hbm, zeros_hbm, out_hbm, idx_v, ones_v, deg_sp):
    cid = lax.axis_index("c")
    sid = lax.axis_index("s")
    wid = _worker_id()
    pltpu.sync_copy(zeros_hbm, deg_sp.at[pl.ds(sid * ROWS_T, ROWS_T)])
    pltpu.sync_copy(ones_hbm, ones_v)
    plsc.subcore_barrier()

    base = wid * EDGES_W

    def body(j, carry):
        pltpu.sync_copy(dst_hbm.at[pl.ds(base + j * C, C)], idx_v)
        pltpu.sync_copy(ones_v, deg_sp.at[idx_v], add=True)
        return carry

    lax.fori_loop(0, CHUNKS_W, body, 0, unroll=False)
    plsc.subcore_barrier()
    pltpu.sync_copy(
        deg_sp.at[pl.ds(sid * ROWS_T, ROWS_T)],
        out_hbm.at[cid, pl.ds(sid * ROWS_T, ROWS_T)],
    )


def _agg_body(y_hbm, src_hbm, dst_hbm, zeros_hbm, out_hbm, si_v, di_v, rows_v, acc_sp):
    cid = lax.axis_index("c")
    sid = lax.axis_index("s")
    wid = _worker_id()
    pltpu.sync_copy(zeros_hbm, acc_sp.at[pl.ds(sid * ROWS_T, ROWS_T)])
    plsc.subcore_barrier()

    base = wid * EDGES_W

    def body(j, carry):
        off = base + j * C
        pltpu.sync_copy(src_hbm.at[pl.ds(off, C)], si_v)
        pltpu.sync_copy(dst_hbm.at[pl.ds(off, C)], di_v)
        pltpu.sync_copy(y_hbm.at[si_v], rows_v)
        pltpu.sync_copy(rows_v, acc_sp.at[di_v], add=True)
        return carry

    lax.fori_loop(0, CHUNKS_W, body, 0, unroll=False)
    plsc.subcore_barrier()
    pltpu.sync_copy(
        acc_sp.at[pl.ds(sid * ROWS_T, ROWS_T)],
        out_hbm.at[cid, pl.ds(sid * ROWS_T, ROWS_T)],
    )


def _sagg_body(y2_hbm, src_hbm, dst_hbm, zeros_hbm, out_hbm, si_v, di_v, vals_v, acc_sp):
    cid = lax.axis_index("c")
    sid = lax.axis_index("s")
    wid = _worker_id()
    pltpu.sync_copy(zeros_hbm, acc_sp.at[pl.ds(sid * ROWS_T, ROWS_T)])
    plsc.subcore_barrier()

    base = wid * EDGES_W

    def body(j, carry):
        off = base + j * C
        pltpu.sync_copy(src_hbm.at[pl.ds(off, C)], si_v)
        pltpu.sync_copy(dst_hbm.at[pl.ds(off, C)], di_v)
        pltpu.sync_copy(y2_hbm.at[si_v], vals_v)
        pltpu.sync_copy(vals_v, acc_sp.at[di_v], add=True)
        return carry

    lax.fori_loop(0, CHUNKS_W, body, 0, unroll=False)
    plsc.subcore_barrier()
    pltpu.sync_copy(
        acc_sp.at[pl.ds(sid * ROWS_T, ROWS_T)],
        out_hbm.at[cid, pl.ds(sid * ROWS_T, ROWS_T)],
    )


@functools.cache
def _sc_kernels():
    mesh = plsc.VectorSubcoreMesh(
        core_axis_name="c", subcore_axis_name="s", num_cores=NC, num_subcores=NS
    )
    deg = pl.kernel(
        _deg_body,
        out_type=jax.ShapeDtypeStruct((NC, N_ACC), jnp.float32),
        mesh=mesh,
        scratch_types=[
            pltpu.VMEM((C,), jnp.int32),
            pltpu.VMEM((C,), jnp.float32),
            pltpu.VMEM_SHARED((N_ACC,), jnp.float32),
        ],
    )
    agg = pl.kernel(
        _agg_body,
        out_type=jax.ShapeDtypeStruct((NC, N_ACC, D), jnp.float32),
        mesh=mesh,
        scratch_types=[
            pltpu.VMEM((C,), jnp.int32),
            pltpu.VMEM((C,), jnp.int32),
            pltpu.VMEM((C, D), jnp.float32),
            pltpu.VMEM_SHARED((N_ACC, D), jnp.float32),
        ],
    )
    sagg = pl.kernel(
        _sagg_body,
        out_type=jax.ShapeDtypeStruct((NC, N_ACC), jnp.float32),
        mesh=mesh,
        scratch_types=[
            pltpu.VMEM((C,), jnp.int32),
            pltpu.VMEM((C,), jnp.int32),
            pltpu.VMEM((C,), jnp.float32),
            pltpu.VMEM_SHARED((N_ACC,), jnp.float32),
        ],
    )
    return deg, agg, sagg


def _yd_body(x_ref, w1_ref, dp0_ref, dp1_ref, y_ref, d_ref):
    d = lax.rsqrt(1.0 + dp0_ref[...] + dp1_ref[...])
    xw = jnp.dot(x_ref[...], w1_ref[...], preferred_element_type=jnp.float32)
    y_ref[...] = xw * d
    d_ref[...] = d


def _yd_tc(x, w1, dp0, dp1):
    return pl.pallas_call(
        _yd_body,
        grid=(NBLK,),
        in_specs=[
            pl.BlockSpec((RB, D), lambda i: (i, 0)),
            pl.BlockSpec((D, D), lambda i: (0, 0)),
            pl.BlockSpec((RB, 1), lambda i: (i, 0)),
            pl.BlockSpec((RB, 1), lambda i: (i, 0)),
        ],
        out_specs=[
            pl.BlockSpec((RB, D), lambda i: (i, 0)),
            pl.BlockSpec((RB, 1), lambda i: (i, 0)),
        ],
        out_shape=[
            jax.ShapeDtypeStruct((N, D), jnp.float32),
            jax.ShapeDtypeStruct((N, 1), jnp.float32),
        ],
    )(x, w1, dp0, dp1)


def _h_body(a0_ref, a1_ref, y_ref, d_ref, b1_ref, w2_ref, lw_ref, y2_ref):
    d = d_ref[...]
    pre = d * (a0_ref[...] + a1_ref[...] + y_ref[...]) + b1_ref[...]
    h1 = jnp.maximum(pre, 0.0)
    wv = jnp.dot(w2_ref[...], lw_ref[...], preferred_element_type=jnp.float32)
    s = jnp.dot(h1, wv, preferred_element_type=jnp.float32)
    y2_ref[...] = d * s


def _h_tc(a0, a1, y, d, b1r, w2, lw):
    return pl.pallas_call(
        _h_body,
        grid=(NBLK,),
        in_specs=[
            pl.BlockSpec((RB, D), lambda i: (i, 0)),
            pl.BlockSpec((RB, D), lambda i: (i, 0)),
            pl.BlockSpec((RB, D), lambda i: (i, 0)),
            pl.BlockSpec((RB, 1), lambda i: (i, 0)),
            pl.BlockSpec((1, D), lambda i: (0, 0)),
            pl.BlockSpec((D, D), lambda i: (0, 0)),
            pl.BlockSpec((D, 1), lambda i: (0, 0)),
        ],
        out_specs=pl.BlockSpec((RB, 1), lambda i: (i, 0)),
        out_shape=jax.ShapeDtypeStruct((N, 1), jnp.float32),
    )(a0, a1, y, d, b1r, w2, lw)


def _pool_body(a0_ref, a1_ref, y2_ref, d_ref, batch_ref, b2_ref, lw_ref,
               lb_ref, out_ref, sums_s, cnts_s):
    i = pl.program_id(0)
    d = d_ref[...]
    c2 = jnp.dot(b2_ref[...], lw_ref[...], preferred_element_type=jnp.float32)
    t = d * (a0_ref[...] + a1_ref[...]) + d * y2_ref[...] + c2
    gid = lax.broadcasted_iota(jnp.int32, (RB, G), 1)
    onehot = (batch_ref[...] == gid).astype(jnp.float32)
    bsum = lax.dot_general(onehot, t, (((0,), (0,)), ((), ())),
                           preferred_element_type=jnp.float32)
    ones = jnp.ones((RB, 1), jnp.float32)
    bcnt = lax.dot_general(onehot, ones, (((0,), (0,)), ((), ())),
                           preferred_element_type=jnp.float32)

    @pl.when(i == 0)
    def _():
        sums_s[...] = jnp.zeros_like(sums_s)
        cnts_s[...] = jnp.zeros_like(cnts_s)

    sums_s[...] += bsum
    cnts_s[...] += bcnt

    @pl.when(i == NBLK - 1)
    def _():
        out_ref[...] = sums_s[...] / jnp.maximum(cnts_s[...], 1.0) + lb_ref[...]


def _pool_tc(a0, a1, y2, d, batch_col, b2r, lw, lbr):
    return pl.pallas_call(
        _pool_body,
        grid=(NBLK,),
        in_specs=[
            pl.BlockSpec((RB, 1), lambda i: (i, 0)),
            pl.BlockSpec((RB, 1), lambda i: (i, 0)),
            pl.BlockSpec((RB, 1), lambda i: (i, 0)),
            pl.BlockSpec((RB, 1), lambda i: (i, 0)),
            pl.BlockSpec((RB, 1), lambda i: (i, 0)),
            pl.BlockSpec((1, D), lambda i: (0, 0)),
            pl.BlockSpec((D, 1), lambda i: (0, 0)),
            pl.BlockSpec((1, 1), lambda i: (0, 0)),
        ],
        out_specs=pl.BlockSpec((G, 1), lambda i: (0, 0)),
        out_shape=jax.ShapeDtypeStruct((G, 1), jnp.float32),
        scratch_shapes=[
            pltpu.VMEM((G, 1), jnp.float32),
            pltpu.VMEM((G, 1), jnp.float32),
        ],
    )(a0, a1, y2, d, batch_col, b2r, lw, lbr)


def kernel(x, edge_index, batch, W1, b1, W2, b2, lin_W, lin_b):
    src = edge_index[0].astype(jnp.int32)
    dst = edge_index[1].astype(jnp.int32)
    pad = E_PAD - E
    src_p = jnp.concatenate([src, jnp.zeros((pad,), jnp.int32)])
    dst_p = jnp.concatenate([dst, jnp.full((pad,), N, jnp.int32)])

    ones_c = jnp.ones((C,), jnp.float32)
    zeros_1d = jnp.zeros((ROWS_T,), jnp.float32)
    zeros_2d = jnp.zeros((ROWS_T, D), jnp.float32)

    _deg_sc, _agg_sc, _sagg_sc = _sc_kernels()

    degp = _deg_sc(dst_p, ones_c, zeros_1d)
    dp0 = degp[0, :N, None]
    dp1 = degp[1, :N, None]

    y, d = _yd_tc(x, W1, dp0, dp1)

    accp = _agg_sc(y, src_p, dst_p, zeros_2d)

    y2 = _h_tc(accp[0, :N], accp[1, :N], y, d,
               b1.reshape(1, D), W2, lin_W)

    acc2p = _sagg_sc(y2.reshape(N), src_p, dst_p, zeros_1d)

    out = _pool_tc(acc2p[0, :N, None], acc2p[1, :N, None], y2, d,
                   batch.astype(jnp.int32).reshape(N, 1),
                   b2.reshape(1, D), lin_W, lin_b.reshape(1, 1))
    return out.reshape(G)


# speedup vs baseline: 14.4265x; 1.0005x over previous
import functools

import jax
import jax.numpy as jnp
from jax import lax
from jax.experimental import pallas as pl
from jax.experimental.pallas import tpu as pltpu
from jax.experimental.pallas import tpu_sc as plsc

N = 10000
E = 320000
D = 128
G = 256
NC = 2
NS = 16
NW = NC * NS
C = 128
CHUNKS_W = 80
E_PAD = NW * CHUNKS_W * C
EDGES_W = CHUNKS_W * C
N_ACC = NW * 320
ROWS_T = N_ACC // NS
RB = 400
NBLK = N // RB

def _worker_id():
    return lax.axis_index("s") * NC + lax.axis_index("c")


def _deg_body(dst2_hbm, ones_hbm, zeros_hbm, out_hbm, di_v, ones_v, sem, deg_sp):
    cid = lax.axis_index("c")
    sid = lax.axis_index("s")
    wid = _worker_id()
    pltpu.sync_copy(zeros_hbm, deg_sp.at[pl.ds(sid * ROWS_T, ROWS_T)])
    pltpu.sync_copy(ones_hbm, ones_v)
    pltpu.sync_copy(dst2_hbm.at[pl.ds(wid * CHUNKS_W, CHUNKS_W)], di_v)
    plsc.subcore_barrier()

    def fire(j, carry):
        pltpu.async_copy(ones_v, deg_sp.at[di_v.at[j, 0]], sem, add=True)
        return carry

    def drain(j, carry):
        pltpu.make_async_copy(ones_v, deg_sp.at[di_v.at[0, 0]], sem).wait()
        return carry

    lax.fori_loop(0, CHUNKS_W, fire, 0, unroll=False)
    lax.fori_loop(0, CHUNKS_W, drain, 0, unroll=False)
    plsc.subcore_barrier()
    pltpu.sync_copy(
        deg_sp.at[pl.ds(sid * ROWS_T, ROWS_T)],
        out_hbm.at[cid, pl.ds(sid * ROWS_T, ROWS_T)],
    )


def _agg_body(y_hbm, src2_hbm, dst2_hbm, zeros_hbm, out_hbm,
              si0, si1, di0, di1, r0, r1,
              g0, g1, s0, s1, i0, i1, d0, d1, acc_sp):
    cid = lax.axis_index("c")
    sid = lax.axis_index("s")
    wid = _worker_id()
    pltpu.sync_copy(zeros_hbm, acc_sp.at[pl.ds(sid * ROWS_T, ROWS_T)])
    plsc.subcore_barrier()

    base = wid * EDGES_W
    last = CHUNKS_W - 1

    pltpu.async_copy(src2_hbm.at[pl.ds(base, C)], si0, i0)
    pltpu.async_copy(dst2_hbm.at[pl.ds(base, C)], di0, d0)
    pltpu.async_copy(src2_hbm.at[pl.ds(base + C, C)], si1, i1)
    pltpu.async_copy(dst2_hbm.at[pl.ds(base + C, C)], di1, d1)
    pltpu.make_async_copy(src2_hbm.at[pl.ds(base, C)], si0, i0).wait()
    pltpu.async_copy(y_hbm.at[si0], r0, g0)
    pltpu.make_async_copy(src2_hbm.at[pl.ds(base, C)], si1, i1).wait()
    pltpu.async_copy(y_hbm.at[si1], r1, g1)

    def body(i, carry):
        j0 = 2 * i
        j1 = 2 * i + 1
        pltpu.make_async_copy(y_hbm.at[si0], r0, g0).wait()
        pltpu.make_async_copy(dst2_hbm.at[pl.ds(base, C)], di0, d0).wait()
        pltpu.async_copy(r0, acc_sp.at[di0], s0, add=True)
        pltpu.make_async_copy(y_hbm.at[si1], r1, g1).wait()
        pltpu.make_async_copy(dst2_hbm.at[pl.ds(base, C)], di1, d1).wait()
        pltpu.async_copy(r1, acc_sp.at[di1], s1, add=True)
        pltpu.make_async_copy(r0, acc_sp.at[di0], s0).wait()

        @pl.when(j0 + 2 <= last)
        def _():
            off = base + (j0 + 2) * C
            pltpu.async_copy(src2_hbm.at[pl.ds(off, C)], si0, i0)
            pltpu.async_copy(dst2_hbm.at[pl.ds(off, C)], di0, d0)
            pltpu.make_async_copy(src2_hbm.at[pl.ds(base, C)], si0, i0).wait()
            pltpu.async_copy(y_hbm.at[si0], r0, g0)

        pltpu.make_async_copy(r1, acc_sp.at[di1], s1).wait()

        @pl.when(j1 + 2 <= last)
        def _():
            off = base + (j1 + 2) * C
            pltpu.async_copy(src2_hbm.at[pl.ds(off, C)], si1, i1)
            pltpu.async_copy(dst2_hbm.at[pl.ds(off, C)], di1, d1)
            pltpu.make_async_copy(src2_hbm.at[pl.ds(base, C)], si1, i1).wait()
            pltpu.async_copy(y_hbm.at[si1], r1, g1)

        return carry

    lax.fori_loop(0, CHUNKS_W // 2, body, 0, unroll=False)
    plsc.subcore_barrier()
    pltpu.sync_copy(
        acc_sp.at[pl.ds(sid * ROWS_T, ROWS_T)],
        out_hbm.at[cid, pl.ds(sid * ROWS_T, ROWS_T)],
    )


def _sagg_body(y2_hbm, src2_hbm, dst2_hbm, zeros_hbm, out_hbm,
               si_v, di_v, vals_v, gsem, ssem, acc_sp):
    cid = lax.axis_index("c")
    sid = lax.axis_index("s")
    wid = _worker_id()
    pltpu.sync_copy(zeros_hbm, acc_sp.at[pl.ds(sid * ROWS_T, ROWS_T)])
    pltpu.sync_copy(src2_hbm.at[pl.ds(wid * CHUNKS_W, CHUNKS_W)], si_v)
    pltpu.sync_copy(dst2_hbm.at[pl.ds(wid * CHUNKS_W, CHUNKS_W)], di_v)
    plsc.subcore_barrier()

    def fire_gather(j, carry):
        pltpu.async_copy(y2_hbm.at[si_v.at[j, 0]], vals_v.at[j, 0], gsem)
        return carry

    def drain_gather(j, carry):
        pltpu.make_async_copy(y2_hbm.at[si_v.at[0, 0]], vals_v.at[0, 0], gsem).wait()
        return carry

    def fire_scatter(j, carry):
        pltpu.async_copy(vals_v.at[j, 0], acc_sp.at[di_v.at[j, 0]], ssem, add=True)
        return carry

    def drain_scatter(j, carry):
        pltpu.make_async_copy(vals_v.at[0, 0], acc_sp.at[di_v.at[0, 0]], ssem).wait()
        return carry

    lax.fori_loop(0, CHUNKS_W, fire_gather, 0, unroll=False)
    lax.fori_loop(0, CHUNKS_W, drain_gather, 0, unroll=False)
    lax.fori_loop(0, CHUNKS_W, fire_scatter, 0, unroll=False)
    lax.fori_loop(0, CHUNKS_W, drain_scatter, 0, unroll=False)
    plsc.subcore_barrier()
    pltpu.sync_copy(
        acc_sp.at[pl.ds(sid * ROWS_T, ROWS_T)],
        out_hbm.at[cid, pl.ds(sid * ROWS_T, ROWS_T)],
    )


@functools.cache
def _sc_kernels():
    mesh = plsc.VectorSubcoreMesh(
        core_axis_name="c", subcore_axis_name="s", num_cores=NC, num_subcores=NS
    )
    deg = pl.kernel(
        _deg_body,
        out_type=jax.ShapeDtypeStruct((NC, N_ACC), jnp.float32),
        mesh=mesh,
        scratch_types=[
            pltpu.VMEM((CHUNKS_W, 1, C), jnp.int32),
            pltpu.VMEM((C,), jnp.float32),
            pltpu.SemaphoreType.DMA,
            pltpu.VMEM_SHARED((N_ACC,), jnp.float32),
        ],
    )
    agg = pl.kernel(
        _agg_body,
        out_type=jax.ShapeDtypeStruct((NC, N_ACC, D), jnp.float32),
        mesh=mesh,
        scratch_types=[
            pltpu.VMEM((C,), jnp.int32),
            pltpu.VMEM((C,), jnp.int32),
            pltpu.VMEM((C,), jnp.int32),
            pltpu.VMEM((C,), jnp.int32),
            pltpu.VMEM((C, D), jnp.float32),
            pltpu.VMEM((C, D), jnp.float32),
            pltpu.SemaphoreType.DMA,
            pltpu.SemaphoreType.DMA,
            pltpu.SemaphoreType.DMA,
            pltpu.SemaphoreType.DMA,
            pltpu.SemaphoreType.DMA,
            pltpu.SemaphoreType.DMA,
            pltpu.SemaphoreType.DMA,
            pltpu.SemaphoreType.DMA,
            pltpu.VMEM_SHARED((N_ACC, D), jnp.float32),
        ],
    )
    sagg = pl.kernel(
        _sagg_body,
        out_type=jax.ShapeDtypeStruct((NC, N_ACC), jnp.float32),
        mesh=mesh,
        scratch_types=[
            pltpu.VMEM((CHUNKS_W, 1, C), jnp.int32),
            pltpu.VMEM((CHUNKS_W, 1, C), jnp.int32),
            pltpu.VMEM((CHUNKS_W, 1, C), jnp.float32),
            pltpu.SemaphoreType.DMA,
            pltpu.SemaphoreType.DMA,
            pltpu.VMEM_SHARED((N_ACC,), jnp.float32),
        ],
    )
    return deg, agg, sagg


def _yd_body(x_ref, w1_ref, dp0_ref, dp1_ref, y_ref, d_ref):
    d = lax.rsqrt(1.0 + dp0_ref[...] + dp1_ref[...])
    xw = jnp.dot(x_ref[...], w1_ref[...], preferred_element_type=jnp.float32)
    y_ref[...] = xw * d
    d_ref[...] = d


def _yd_tc(x, w1, dp0, dp1):
    return pl.pallas_call(
        _yd_body,
        grid=(NBLK,),
        in_specs=[
            pl.BlockSpec((RB, D), lambda i: (i, 0)),
            pl.BlockSpec((D, D), lambda i: (0, 0)),
            pl.BlockSpec((RB, 1), lambda i: (i, 0)),
            pl.BlockSpec((RB, 1), lambda i: (i, 0)),
        ],
        out_specs=[
            pl.BlockSpec((RB, D), lambda i: (i, 0)),
            pl.BlockSpec((RB, 1), lambda i: (i, 0)),
        ],
        out_shape=[
            jax.ShapeDtypeStruct((N, D), jnp.float32),
            jax.ShapeDtypeStruct((N, 1), jnp.float32),
        ],
    )(x, w1, dp0, dp1)


def _h_body(a0_ref, a1_ref, y_ref, d_ref, b1_ref, w2_ref, lw_ref, y2_ref):
    d = d_ref[...]
    pre = d * (a0_ref[...] + a1_ref[...] + y_ref[...]) + b1_ref[...]
    h1 = jnp.maximum(pre, 0.0)
    wv = jnp.dot(w2_ref[...], lw_ref[...], preferred_element_type=jnp.float32)
    s = jnp.dot(h1, wv, preferred_element_type=jnp.float32)
    y2_ref[...] = d * s


def _h_tc(a0, a1, y, d, b1r, w2, lw):
    return pl.pallas_call(
        _h_body,
        grid=(NBLK,),
        in_specs=[
            pl.BlockSpec((RB, D), lambda i: (i, 0)),
            pl.BlockSpec((RB, D), lambda i: (i, 0)),
            pl.BlockSpec((RB, D), lambda i: (i, 0)),
            pl.BlockSpec((RB, 1), lambda i: (i, 0)),
            pl.BlockSpec((1, D), lambda i: (0, 0)),
            pl.BlockSpec((D, D), lambda i: (0, 0)),
            pl.BlockSpec((D, 1), lambda i: (0, 0)),
        ],
        out_specs=pl.BlockSpec((RB, 1), lambda i: (i, 0)),
        out_shape=jax.ShapeDtypeStruct((N, 1), jnp.float32),
    )(a0, a1, y, d, b1r, w2, lw)


def _pool_body(a0_ref, a1_ref, y2_ref, d_ref, batch_ref, b2_ref, lw_ref,
               lb_ref, out_ref, sums_s, cnts_s):
    i = pl.program_id(0)
    d = d_ref[...]
    c2 = jnp.dot(b2_ref[...], lw_ref[...], preferred_element_type=jnp.float32)
    t = d * (a0_ref[...] + a1_ref[...]) + d * y2_ref[...] + c2
    gid = lax.broadcasted_iota(jnp.int32, (RB, G), 1)
    onehot = (batch_ref[...] == gid).astype(jnp.float32)
    bsum = lax.dot_general(onehot, t, (((0,), (0,)), ((), ())),
                           preferred_element_type=jnp.float32)
    ones = jnp.ones((RB, 1), jnp.float32)
    bcnt = lax.dot_general(onehot, ones, (((0,), (0,)), ((), ())),
                           preferred_element_type=jnp.float32)

    @pl.when(i == 0)
    def _():
        sums_s[...] = jnp.zeros_like(sums_s)
        cnts_s[...] = jnp.zeros_like(cnts_s)

    sums_s[...] += bsum
    cnts_s[...] += bcnt

    @pl.when(i == NBLK - 1)
    def _():
        out_ref[...] = sums_s[...] / jnp.maximum(cnts_s[...], 1.0) + lb_ref[...]


def _pool_tc(a0, a1, y2, d, batch_col, b2r, lw, lbr):
    return pl.pallas_call(
        _pool_body,
        grid=(NBLK,),
        in_specs=[
            pl.BlockSpec((RB, 1), lambda i: (i, 0)),
            pl.BlockSpec((RB, 1), lambda i: (i, 0)),
            pl.BlockSpec((RB, 1), lambda i: (i, 0)),
            pl.BlockSpec((RB, 1), lambda i: (i, 0)),
            pl.BlockSpec((RB, 1), lambda i: (i, 0)),
            pl.BlockSpec((1, D), lambda i: (0, 0)),
            pl.BlockSpec((D, 1), lambda i: (0, 0)),
            pl.BlockSpec((1, 1), lambda i: (0, 0)),
        ],
        out_specs=pl.BlockSpec((G, 1), lambda i: (0, 0)),
        out_shape=jax.ShapeDtypeStruct((G, 1), jnp.float32),
        scratch_shapes=[
            pltpu.VMEM((G, 1), jnp.float32),
            pltpu.VMEM((G, 1), jnp.float32),
        ],
    )(a0, a1, y2, d, batch_col, b2r, lw, lbr)


def kernel(x, edge_index, batch, W1, b1, W2, b2, lin_W, lin_b):
    src = edge_index[0].astype(jnp.int32)
    dst = edge_index[1].astype(jnp.int32)
    pad = E_PAD - E
    src_f = jnp.concatenate([src, jnp.zeros((pad,), jnp.int32)])
    dst_pad = N + (jnp.arange(pad, dtype=jnp.int32) % (N_ACC - N - 1))
    dst_f = jnp.concatenate([dst, dst_pad])
    src_p = src_f.reshape(E_PAD // C, 1, C)
    dst_p = dst_f.reshape(E_PAD // C, 1, C)

    ones_c = jnp.ones((C,), jnp.float32)
    zeros_1d = jnp.zeros((ROWS_T,), jnp.float32)
    zeros_2d = jnp.zeros((ROWS_T, D), jnp.float32)

    _deg_sc, _agg_sc, _sagg_sc = _sc_kernels()

    degp = _deg_sc(dst_p, ones_c, zeros_1d)
    dp0 = degp[0, :N, None]
    dp1 = degp[1, :N, None]

    y, d = _yd_tc(x, W1, dp0, dp1)

    accp = _agg_sc(y, src_f, dst_f, zeros_2d)

    y2 = _h_tc(accp[0, :N], accp[1, :N], y, d,
               b1.reshape(1, D), W2, lin_W)

    acc2p = _sagg_sc(y2.reshape(N), src_p, dst_p, zeros_1d)

    out = _pool_tc(acc2p[0, :N, None], acc2p[1, :N, None], y2, d,
                   batch.astype(jnp.int32).reshape(N, 1),
                   b2.reshape(1, D), lin_W, lin_b.reshape(1, 1))
    return out.reshape(G)


# speedup vs baseline: 30.9361x; 2.1444x over previous
import functools

import jax
import jax.numpy as jnp
from jax import lax
from jax.experimental import pallas as pl
from jax.experimental.pallas import tpu as pltpu
from jax.experimental.pallas import tpu_sc as plsc

N = 10000
E = 320000
D = 128
G = 256
NC = 2
NS = 16
NW = NC * NS
C = 128
CHUNKS_W = 80
E_PAD = NW * CHUNKS_W * C
EDGES_W = CHUNKS_W * C
N_ACC = NW * 320
ROWS_T = N_ACC // NS
CHUNKS_T = E_PAD // (NS * C)
EDGES_T = CHUNKS_T * C
DH = D // 2
RB = 400
NBLK = N // RB

def _worker_id():
    return lax.axis_index("s") * NC + lax.axis_index("c")


def _deg_body(dst2_hbm, ones_hbm, zeros_hbm, out_hbm, di_v, ones_v, sem, deg_sp):
    cid = lax.axis_index("c")
    sid = lax.axis_index("s")
    wid = _worker_id()
    pltpu.sync_copy(zeros_hbm, deg_sp.at[pl.ds(sid * ROWS_T, ROWS_T)])
    pltpu.sync_copy(ones_hbm, ones_v)
    pltpu.sync_copy(dst2_hbm.at[pl.ds(wid * CHUNKS_W, CHUNKS_W)], di_v)
    plsc.subcore_barrier()

    def fire(j, carry):
        pltpu.async_copy(ones_v, deg_sp.at[di_v.at[j, 0]], sem, add=True)
        return carry

    def drain(j, carry):
        pltpu.make_async_copy(ones_v, deg_sp.at[di_v.at[0, 0]], sem).wait()
        return carry

    lax.fori_loop(0, CHUNKS_W, fire, 0, unroll=False)
    lax.fori_loop(0, CHUNKS_W, drain, 0, unroll=False)
    plsc.subcore_barrier()
    pltpu.sync_copy(
        deg_sp.at[pl.ds(sid * ROWS_T, ROWS_T)],
        out_hbm.at[cid, pl.ds(sid * ROWS_T, ROWS_T)],
    )


def _agg_body(y_hbm, src2_hbm, dst2_hbm, zeros_hbm, out_hbm,
              si0, si1, di0, di1, r0, r1,
              g0, g1, s0, s1, i0, i1, d0, d1, acc_sp):
    cid = lax.axis_index("c")
    sid = lax.axis_index("s")
    wid = _worker_id()
    pltpu.sync_copy(zeros_hbm, acc_sp.at[pl.ds(sid * ROWS_T, ROWS_T)])
    plsc.subcore_barrier()

    base = wid * EDGES_W
    last = CHUNKS_W - 1

    pltpu.async_copy(src2_hbm.at[pl.ds(base, C)], si0, i0)
    pltpu.async_copy(dst2_hbm.at[pl.ds(base, C)], di0, d0)
    pltpu.async_copy(src2_hbm.at[pl.ds(base + C, C)], si1, i1)
    pltpu.async_copy(dst2_hbm.at[pl.ds(base + C, C)], di1, d1)
    pltpu.make_async_copy(src2_hbm.at[pl.ds(base, C)], si0, i0).wait()
    pltpu.async_copy(y_hbm.at[si0], r0, g0)
    pltpu.make_async_copy(src2_hbm.at[pl.ds(base, C)], si1, i1).wait()
    pltpu.async_copy(y_hbm.at[si1], r1, g1)

    def body(i, carry):
        j0 = 2 * i
        j1 = 2 * i + 1
        pltpu.make_async_copy(y_hbm.at[si0], r0, g0).wait()
        pltpu.make_async_copy(dst2_hbm.at[pl.ds(base, C)], di0, d0).wait()
        pltpu.async_copy(r0, acc_sp.at[di0], s0, add=True)
        pltpu.make_async_copy(y_hbm.at[si1], r1, g1).wait()
        pltpu.make_async_copy(dst2_hbm.at[pl.ds(base, C)], di1, d1).wait()
        pltpu.async_copy(r1, acc_sp.at[di1], s1, add=True)
        pltpu.make_async_copy(r0, acc_sp.at[di0], s0).wait()

        @pl.when(j0 + 2 <= last)
        def _():
            off = base + (j0 + 2) * C
            pltpu.async_copy(src2_hbm.at[pl.ds(off, C)], si0, i0)
            pltpu.async_copy(dst2_hbm.at[pl.ds(off, C)], di0, d0)
            pltpu.make_async_copy(src2_hbm.at[pl.ds(base, C)], si0, i0).wait()
            pltpu.async_copy(y_hbm.at[si0], r0, g0)

        pltpu.make_async_copy(r1, acc_sp.at[di1], s1).wait()

        @pl.when(j1 + 2 <= last)
        def _():
            off = base + (j1 + 2) * C
            pltpu.async_copy(src2_hbm.at[pl.ds(off, C)], si1, i1)
            pltpu.async_copy(dst2_hbm.at[pl.ds(off, C)], di1, d1)
            pltpu.make_async_copy(src2_hbm.at[pl.ds(base, C)], si1, i1).wait()
            pltpu.async_copy(y_hbm.at[si1], r1, g1)

        return carry

    lax.fori_loop(0, CHUNKS_W // 2, body, 0, unroll=False)
    plsc.subcore_barrier()
    pltpu.sync_copy(
        acc_sp.at[pl.ds(sid * ROWS_T, ROWS_T)],
        out_hbm.at[cid, pl.ds(sid * ROWS_T, ROWS_T)],
    )


def _sagg_body(y2_hbm, src2_hbm, dst2_hbm, zeros_hbm, out_hbm,
               si_v, di_v, vals_v, gsem, ssem, acc_sp):
    cid = lax.axis_index("c")
    sid = lax.axis_index("s")
    wid = _worker_id()
    pltpu.sync_copy(zeros_hbm, acc_sp.at[pl.ds(sid * ROWS_T, ROWS_T)])
    pltpu.sync_copy(src2_hbm.at[pl.ds(wid * CHUNKS_W, CHUNKS_W)], si_v)
    pltpu.sync_copy(dst2_hbm.at[pl.ds(wid * CHUNKS_W, CHUNKS_W)], di_v)
    plsc.subcore_barrier()

    def fire_gather(j, carry):
        pltpu.async_copy(y2_hbm.at[si_v.at[j, 0]], vals_v.at[j, 0], gsem)
        return carry

    def drain_gather(j, carry):
        pltpu.make_async_copy(y2_hbm.at[si_v.at[0, 0]], vals_v.at[0, 0], gsem).wait()
        return carry

    def fire_scatter(j, carry):
        pltpu.async_copy(vals_v.at[j, 0], acc_sp.at[di_v.at[j, 0]], ssem, add=True)
        return carry

    def drain_scatter(j, carry):
        pltpu.make_async_copy(vals_v.at[0, 0], acc_sp.at[di_v.at[0, 0]], ssem).wait()
        return carry

    lax.fori_loop(0, CHUNKS_W, fire_gather, 0, unroll=False)
    lax.fori_loop(0, CHUNKS_W, drain_gather, 0, unroll=False)
    lax.fori_loop(0, CHUNKS_W, fire_scatter, 0, unroll=False)
    lax.fori_loop(0, CHUNKS_W, drain_scatter, 0, unroll=False)
    plsc.subcore_barrier()
    pltpu.sync_copy(
        acc_sp.at[pl.ds(sid * ROWS_T, ROWS_T)],
        out_hbm.at[cid, pl.ds(sid * ROWS_T, ROWS_T)],
    )


@functools.cache
def _sc_kernels():
    mesh = plsc.VectorSubcoreMesh(
        core_axis_name="c", subcore_axis_name="s", num_cores=NC, num_subcores=NS
    )
    deg = pl.kernel(
        _deg_body,
        out_type=jax.ShapeDtypeStruct((NC, N_ACC), jnp.float32),
        mesh=mesh,
        scratch_types=[
            pltpu.VMEM((CHUNKS_W, 1, C), jnp.int32),
            pltpu.VMEM((C,), jnp.float32),
            pltpu.SemaphoreType.DMA,
            pltpu.VMEM_SHARED((N_ACC,), jnp.float32),
        ],
    )
    agg = pl.kernel(
        _agg_body,
        out_type=jax.ShapeDtypeStruct((NC, N_ACC, D), jnp.float32),
        mesh=mesh,
        scratch_types=[
            pltpu.VMEM((C,), jnp.int32),
            pltpu.VMEM((C,), jnp.int32),
            pltpu.VMEM((C,), jnp.int32),
            pltpu.VMEM((C,), jnp.int32),
            pltpu.VMEM((C, D), jnp.float32),
            pltpu.VMEM((C, D), jnp.float32),
            pltpu.SemaphoreType.DMA,
            pltpu.SemaphoreType.DMA,
            pltpu.SemaphoreType.DMA,
            pltpu.SemaphoreType.DMA,
            pltpu.SemaphoreType.DMA,
            pltpu.SemaphoreType.DMA,
            pltpu.SemaphoreType.DMA,
            pltpu.SemaphoreType.DMA,
            pltpu.VMEM_SHARED((N_ACC, D), jnp.float32),
        ],
    )
    sagg = pl.kernel(
        _sagg_body,
        out_type=jax.ShapeDtypeStruct((NC, N_ACC), jnp.float32),
        mesh=mesh,
        scratch_types=[
            pltpu.VMEM((CHUNKS_W, 1, C), jnp.int32),
            pltpu.VMEM((CHUNKS_W, 1, C), jnp.int32),
            pltpu.VMEM((CHUNKS_W, 1, C), jnp.float32),
            pltpu.SemaphoreType.DMA,
            pltpu.SemaphoreType.DMA,
            pltpu.VMEM_SHARED((N_ACC,), jnp.float32),
        ],
    )
    return deg, agg, sagg


def _yd_body(x_ref, w1_ref, dp0_ref, dp1_ref, y_ref, d_ref):
    d = lax.rsqrt(1.0 + dp0_ref[...] + dp1_ref[...])
    xw = jnp.dot(x_ref[...], w1_ref[...], preferred_element_type=jnp.float32)
    y_ref[...] = xw * d
    d_ref[...] = d


def _yd_tc(x, w1, dp0, dp1):
    return pl.pallas_call(
        _yd_body,
        grid=(NBLK,),
        in_specs=[
            pl.BlockSpec((RB, D), lambda i: (i, 0)),
            pl.BlockSpec((D, D), lambda i: (0, 0)),
            pl.BlockSpec((RB, 1), lambda i: (i, 0)),
            pl.BlockSpec((RB, 1), lambda i: (i, 0)),
        ],
        out_specs=[
            pl.BlockSpec((RB, D), lambda i: (i, 0)),
            pl.BlockSpec((RB, 1), lambda i: (i, 0)),
        ],
        out_shape=[
            jax.ShapeDtypeStruct((N, D), jnp.float32),
            jax.ShapeDtypeStruct((N, 1), jnp.float32),
        ],
    )(x, w1, dp0, dp1)


def _h_body(a0_ref, a1_ref, y_ref, d_ref, b1_ref, w2_ref, lw_ref, y2_ref):
    d = d_ref[...]
    agg = a0_ref[...] + a1_ref[...]
    pre = d * (agg + y_ref[...]) + b1_ref[...]
    h1 = jnp.maximum(pre, 0.0)
    wv = jnp.dot(w2_ref[...], lw_ref[...], preferred_element_type=jnp.float32)
    s = jnp.dot(h1, wv, preferred_element_type=jnp.float32)
    y2_ref[...] = d * s


def _h_tc(a0, a1, y, d, b1r, w2, lw):
    return pl.pallas_call(
        _h_body,
        grid=(NBLK,),
        in_specs=[
            pl.BlockSpec((RB, D), lambda i: (i, 0)),
            pl.BlockSpec((RB, D), lambda i: (i, 0)),
            pl.BlockSpec((RB, D), lambda i: (i, 0)),
            pl.BlockSpec((RB, 1), lambda i: (i, 0)),
            pl.BlockSpec((1, D), lambda i: (0, 0)),
            pl.BlockSpec((D, D), lambda i: (0, 0)),
            pl.BlockSpec((D, 1), lambda i: (0, 0)),
        ],
        out_specs=pl.BlockSpec((RB, 1), lambda i: (i, 0)),
        out_shape=jax.ShapeDtypeStruct((N, 1), jnp.float32),
    )(a0, a1, y, d, b1r, w2, lw)


def _pool_body(a0_ref, a1_ref, y2_ref, d_ref, batch_ref, b2_ref, lw_ref,
               lb_ref, out_ref, sums_s, cnts_s):
    i = pl.program_id(0)
    d = d_ref[...]
    c2 = jnp.dot(b2_ref[...], lw_ref[...], preferred_element_type=jnp.float32)
    t = d * (a0_ref[...] + a1_ref[...]) + d * y2_ref[...] + c2
    gid = lax.broadcasted_iota(jnp.int32, (RB, G), 1)
    onehot = (batch_ref[...] == gid).astype(jnp.float32)
    bsum = lax.dot_general(onehot, t, (((0,), (0,)), ((), ())),
                           preferred_element_type=jnp.float32)
    ones = jnp.ones((RB, 1), jnp.float32)
    bcnt = lax.dot_general(onehot, ones, (((0,), (0,)), ((), ())),
                           preferred_element_type=jnp.float32)

    @pl.when(i == 0)
    def _():
        sums_s[...] = jnp.zeros_like(sums_s)
        cnts_s[...] = jnp.zeros_like(cnts_s)

    sums_s[...] += bsum
    cnts_s[...] += bcnt

    @pl.when(i == NBLK - 1)
    def _():
        out_ref[...] = sums_s[...] / jnp.maximum(cnts_s[...], 1.0) + lb_ref[...]


def _pool_tc(a0, a1, y2, d, batch_col, b2r, lw, lbr):
    return pl.pallas_call(
        _pool_body,
        grid=(NBLK,),
        in_specs=[
            pl.BlockSpec((RB, 1), lambda i: (i, 0)),
            pl.BlockSpec((RB, 1), lambda i: (i, 0)),
            pl.BlockSpec((RB, 1), lambda i: (i, 0)),
            pl.BlockSpec((RB, 1), lambda i: (i, 0)),
            pl.BlockSpec((RB, 1), lambda i: (i, 0)),
            pl.BlockSpec((1, D), lambda i: (0, 0)),
            pl.BlockSpec((D, 1), lambda i: (0, 0)),
            pl.BlockSpec((1, 1), lambda i: (0, 0)),
        ],
        out_specs=pl.BlockSpec((G, 1), lambda i: (0, 0)),
        out_shape=jax.ShapeDtypeStruct((G, 1), jnp.float32),
        scratch_shapes=[
            pltpu.VMEM((G, 1), jnp.float32),
            pltpu.VMEM((G, 1), jnp.float32),
        ],
    )(a0, a1, y2, d, batch_col, b2r, lw, lbr)


def kernel(x, edge_index, batch, W1, b1, W2, b2, lin_W, lin_b):
    src = edge_index[0].astype(jnp.int32)
    dst = edge_index[1].astype(jnp.int32)
    pad = E_PAD - E
    src_pad = jnp.arange(pad, dtype=jnp.int32) % N
    src_f = jnp.concatenate([src, src_pad])
    dst_pad = N + (jnp.arange(pad, dtype=jnp.int32) % (N_ACC - N - 1))
    dst_f = jnp.concatenate([dst, dst_pad])
    src_p = src_f.reshape(E_PAD // C, 1, C)
    dst_p = dst_f.reshape(E_PAD // C, 1, C)

    ones_c = jnp.ones((C,), jnp.float32)
    zeros_1d = jnp.zeros((ROWS_T,), jnp.float32)
    zeros_2d = jnp.zeros((ROWS_T, D), jnp.float32)

    _deg_sc, _agg_sc, _sagg_sc = _sc_kernels()

    degp = _deg_sc(dst_p, ones_c, zeros_1d)
    dp0 = degp[0, :N, None]
    dp1 = degp[1, :N, None]

    y, d = _yd_tc(x, W1, dp0, dp1)

    accp = _agg_sc(y, src_f, dst_f, zeros_2d)

    y2 = _h_tc(accp[0, :N], accp[1, :N], y, d,
               b1.reshape(1, D), W2, lin_W)

    acc2p = _sagg_sc(y2.reshape(N), src_p, dst_p, zeros_1d)

    out = _pool_tc(acc2p[0, :N, None], acc2p[1, :N, None], y2, d,
                   batch.astype(jnp.int32).reshape(N, 1),
                   b2.reshape(1, D), lin_W, lin_b.reshape(1, 1))
    return out.reshape(G)


# speedup vs baseline: 34.7081x; 1.1219x over previous
import functools

import jax
import jax.numpy as jnp
from jax import lax
from jax.experimental import pallas as pl
from jax.experimental.pallas import tpu as pltpu
from jax.experimental.pallas import tpu_sc as plsc

N = 10000
E = 320000
D = 128
G = 256
NC = 2
NS = 16
NW = NC * NS
C = 128
CHUNKS_W = 80
E_PAD = NW * CHUNKS_W * C
EDGES_W = CHUNKS_W * C
N_ACC = NW * 320
ROWS_T = N_ACC // NS
CHUNKS_T = E_PAD // (NS * C)
EDGES_T = CHUNKS_T * C
DH = D // 2
RB = 400
NBLK = N // RB

def _worker_id():
    return lax.axis_index("s") * NC + lax.axis_index("c")


def _deg_body(dst2_hbm, ones_hbm, zeros_hbm, out_hbm, di_v, ones_v, sem, deg_sp):
    cid = lax.axis_index("c")
    sid = lax.axis_index("s")
    wid = _worker_id()
    pltpu.sync_copy(zeros_hbm, deg_sp.at[pl.ds(sid * ROWS_T, ROWS_T)])
    pltpu.sync_copy(ones_hbm, ones_v)
    pltpu.sync_copy(dst2_hbm.at[pl.ds(wid * CHUNKS_W, CHUNKS_W)], di_v)
    plsc.subcore_barrier()

    def fire(j, carry):
        pltpu.async_copy(ones_v, deg_sp.at[di_v.at[j, 0]], sem, add=True)
        return carry

    def drain(j, carry):
        pltpu.make_async_copy(ones_v, deg_sp.at[di_v.at[0, 0]], sem).wait()
        return carry

    lax.fori_loop(0, CHUNKS_W, fire, 0, unroll=False)
    lax.fori_loop(0, CHUNKS_W, drain, 0, unroll=False)
    plsc.subcore_barrier()
    pltpu.sync_copy(
        deg_sp.at[pl.ds(sid * ROWS_T, ROWS_T)],
        out_hbm.at[cid, pl.ds(sid * ROWS_T, ROWS_T)],
    )


def _agg_body(y_hbm, src2_hbm, dst2_hbm, zeros_hbm, out_hbm,
              si0, si1, di0, di1, r0, r1,
              g0, g1, s0, s1, i0, i1, d0, d1, acc_sp):
    cid = lax.axis_index("c")
    sid = lax.axis_index("s")
    wid = _worker_id()
    pltpu.sync_copy(zeros_hbm, acc_sp.at[pl.ds(sid * ROWS_T, ROWS_T)])
    plsc.subcore_barrier()

    base = wid * EDGES_W
    last = CHUNKS_W - 1

    pltpu.async_copy(src2_hbm.at[pl.ds(base, C)], si0, i0)
    pltpu.async_copy(dst2_hbm.at[pl.ds(base, C)], di0, d0)
    pltpu.async_copy(src2_hbm.at[pl.ds(base + C, C)], si1, i1)
    pltpu.async_copy(dst2_hbm.at[pl.ds(base + C, C)], di1, d1)
    pltpu.make_async_copy(src2_hbm.at[pl.ds(base, C)], si0, i0).wait()
    pltpu.async_copy(y_hbm.at[si0], r0, g0)
    pltpu.make_async_copy(src2_hbm.at[pl.ds(base, C)], si1, i1).wait()
    pltpu.async_copy(y_hbm.at[si1], r1, g1)

    def body(i, carry):
        j0 = 2 * i
        j1 = 2 * i + 1
        pltpu.make_async_copy(y_hbm.at[si0], r0, g0).wait()
        pltpu.make_async_copy(dst2_hbm.at[pl.ds(base, C)], di0, d0).wait()
        pltpu.async_copy(r0, acc_sp.at[di0], s0, add=True)
        pltpu.make_async_copy(y_hbm.at[si1], r1, g1).wait()
        pltpu.make_async_copy(dst2_hbm.at[pl.ds(base, C)], di1, d1).wait()
        pltpu.async_copy(r1, acc_sp.at[di1], s1, add=True)
        pltpu.make_async_copy(r0, acc_sp.at[di0], s0).wait()

        @pl.when(j0 + 2 <= last)
        def _():
            off = base + (j0 + 2) * C
            pltpu.async_copy(src2_hbm.at[pl.ds(off, C)], si0, i0)
            pltpu.async_copy(dst2_hbm.at[pl.ds(off, C)], di0, d0)
            pltpu.make_async_copy(src2_hbm.at[pl.ds(base, C)], si0, i0).wait()
            pltpu.async_copy(y_hbm.at[si0], r0, g0)

        pltpu.make_async_copy(r1, acc_sp.at[di1], s1).wait()

        @pl.when(j1 + 2 <= last)
        def _():
            off = base + (j1 + 2) * C
            pltpu.async_copy(src2_hbm.at[pl.ds(off, C)], si1, i1)
            pltpu.async_copy(dst2_hbm.at[pl.ds(off, C)], di1, d1)
            pltpu.make_async_copy(src2_hbm.at[pl.ds(base, C)], si1, i1).wait()
            pltpu.async_copy(y_hbm.at[si1], r1, g1)

        return carry

    lax.fori_loop(0, CHUNKS_W // 2, body, 0, unroll=False)
    plsc.subcore_barrier()
    pltpu.sync_copy(
        acc_sp.at[pl.ds(sid * ROWS_T, ROWS_T)],
        out_hbm.at[cid, pl.ds(sid * ROWS_T, ROWS_T)],
    )


def _sagg_body(y2_hbm, src2_hbm, dst2_hbm, zeros_hbm, out_hbm,
               si_v, di_v, vals_v, y2_t, gsem, ssem, acc_sp):
    cid = lax.axis_index("c")
    sid = lax.axis_index("s")
    wid = _worker_id()
    pltpu.sync_copy(zeros_hbm, acc_sp.at[pl.ds(sid * ROWS_T, ROWS_T)])
    pltpu.sync_copy(y2_hbm, y2_t)
    pltpu.sync_copy(src2_hbm.at[pl.ds(wid * CHUNKS_W, CHUNKS_W)], si_v)
    pltpu.sync_copy(dst2_hbm.at[pl.ds(wid * CHUNKS_W, CHUNKS_W)], di_v)
    plsc.subcore_barrier()

    def gather_fire(j, carry):
        for k in range(C // 16):
            idx16 = si_v[j, 0, pl.ds(k * 16, 16)]
            vals_v[j, 0, pl.ds(k * 16, 16)] = plsc.load_gather(y2_t, [idx16])
        pltpu.async_copy(vals_v.at[j, 0], acc_sp.at[di_v.at[j, 0]], ssem, add=True)
        return carry

    def drain_scatter(j, carry):
        pltpu.make_async_copy(vals_v.at[0, 0], acc_sp.at[di_v.at[0, 0]], ssem).wait()
        return carry

    lax.fori_loop(0, CHUNKS_W, gather_fire, 0, unroll=False)
    lax.fori_loop(0, CHUNKS_W, drain_scatter, 0, unroll=False)
    plsc.subcore_barrier()
    pltpu.sync_copy(
        acc_sp.at[pl.ds(sid * ROWS_T, ROWS_T)],
        out_hbm.at[cid, pl.ds(sid * ROWS_T, ROWS_T)],
    )


@functools.cache
def _sc_kernels():
    mesh = plsc.VectorSubcoreMesh(
        core_axis_name="c", subcore_axis_name="s", num_cores=NC, num_subcores=NS
    )
    deg = pl.kernel(
        _deg_body,
        out_type=jax.ShapeDtypeStruct((NC, N_ACC), jnp.float32),
        mesh=mesh,
        scratch_types=[
            pltpu.VMEM((CHUNKS_W, 1, C), jnp.int32),
            pltpu.VMEM((C,), jnp.float32),
            pltpu.SemaphoreType.DMA,
            pltpu.VMEM_SHARED((N_ACC,), jnp.float32),
        ],
    )
    agg = pl.kernel(
        _agg_body,
        out_type=jax.ShapeDtypeStruct((NC, N_ACC, D), jnp.float32),
        mesh=mesh,
        scratch_types=[
            pltpu.VMEM((C,), jnp.int32),
            pltpu.VMEM((C,), jnp.int32),
            pltpu.VMEM((C,), jnp.int32),
            pltpu.VMEM((C,), jnp.int32),
            pltpu.VMEM((C, D), jnp.float32),
            pltpu.VMEM((C, D), jnp.float32),
            pltpu.SemaphoreType.DMA,
            pltpu.SemaphoreType.DMA,
            pltpu.SemaphoreType.DMA,
            pltpu.SemaphoreType.DMA,
            pltpu.SemaphoreType.DMA,
            pltpu.SemaphoreType.DMA,
            pltpu.SemaphoreType.DMA,
            pltpu.SemaphoreType.DMA,
            pltpu.VMEM_SHARED((N_ACC, D), jnp.float32),
        ],
    )
    sagg = pl.kernel(
        _sagg_body,
        out_type=jax.ShapeDtypeStruct((NC, N_ACC), jnp.float32),
        mesh=mesh,
        compiler_params=pltpu.CompilerParams(needs_layout_passes=False),
        scratch_types=[
            pltpu.VMEM((CHUNKS_W, 1, C), jnp.int32),
            pltpu.VMEM((CHUNKS_W, 1, C), jnp.int32),
            pltpu.VMEM((CHUNKS_W, 1, C), jnp.float32),
            pltpu.VMEM((N_ACC,), jnp.float32),
            pltpu.SemaphoreType.DMA,
            pltpu.SemaphoreType.DMA,
            pltpu.VMEM_SHARED((N_ACC,), jnp.float32),
        ],
    )
    return deg, agg, sagg


def _yd_body(x_ref, w1_ref, dp0_ref, dp1_ref, y_ref, d_ref):
    d = lax.rsqrt(1.0 + dp0_ref[...] + dp1_ref[...])
    xw = jnp.dot(x_ref[...], w1_ref[...], preferred_element_type=jnp.float32)
    y_ref[...] = xw * d
    d_ref[...] = d


def _yd_tc(x, w1, dp0, dp1):
    return pl.pallas_call(
        _yd_body,
        grid=(NBLK,),
        in_specs=[
            pl.BlockSpec((RB, D), lambda i: (i, 0)),
            pl.BlockSpec((D, D), lambda i: (0, 0)),
            pl.BlockSpec((RB, 1), lambda i: (i, 0)),
            pl.BlockSpec((RB, 1), lambda i: (i, 0)),
        ],
        out_specs=[
            pl.BlockSpec((RB, D), lambda i: (i, 0)),
            pl.BlockSpec((RB, 1), lambda i: (i, 0)),
        ],
        out_shape=[
            jax.ShapeDtypeStruct((N, D), jnp.float32),
            jax.ShapeDtypeStruct((N, 1), jnp.float32),
        ],
    )(x, w1, dp0, dp1)


def _h_body(a0_ref, a1_ref, y_ref, d_ref, b1_ref, w2_ref, lw_ref, y2_ref):
    d = d_ref[...]
    agg = a0_ref[...] + a1_ref[...]
    pre = d * (agg + y_ref[...]) + b1_ref[...]
    h1 = jnp.maximum(pre, 0.0)
    wv = jnp.dot(w2_ref[...], lw_ref[...], preferred_element_type=jnp.float32)
    s = jnp.dot(h1, wv, preferred_element_type=jnp.float32)
    y2_ref[...] = d * s


def _h_tc(a0, a1, y, d, b1r, w2, lw):
    return pl.pallas_call(
        _h_body,
        grid=(NBLK,),
        in_specs=[
            pl.BlockSpec((RB, D), lambda i: (i, 0)),
            pl.BlockSpec((RB, D), lambda i: (i, 0)),
            pl.BlockSpec((RB, D), lambda i: (i, 0)),
            pl.BlockSpec((RB, 1), lambda i: (i, 0)),
            pl.BlockSpec((1, D), lambda i: (0, 0)),
            pl.BlockSpec((D, D), lambda i: (0, 0)),
            pl.BlockSpec((D, 1), lambda i: (0, 0)),
        ],
        out_specs=pl.BlockSpec((RB, 1), lambda i: (i, 0)),
        out_shape=jax.ShapeDtypeStruct((N, 1), jnp.float32),
    )(a0, a1, y, d, b1r, w2, lw)


def _pool_body(a0_ref, a1_ref, y2_ref, d_ref, batch_ref, b2_ref, lw_ref,
               lb_ref, out_ref, sums_s, cnts_s):
    i = pl.program_id(0)
    d = d_ref[...]
    c2 = jnp.dot(b2_ref[...], lw_ref[...], preferred_element_type=jnp.float32)
    t = d * (a0_ref[...] + a1_ref[...]) + d * y2_ref[...] + c2
    gid = lax.broadcasted_iota(jnp.int32, (RB, G), 1)
    onehot = (batch_ref[...] == gid).astype(jnp.float32)
    bsum = lax.dot_general(onehot, t, (((0,), (0,)), ((), ())),
                           preferred_element_type=jnp.float32)
    ones = jnp.ones((RB, 1), jnp.float32)
    bcnt = lax.dot_general(onehot, ones, (((0,), (0,)), ((), ())),
                           preferred_element_type=jnp.float32)

    @pl.when(i == 0)
    def _():
        sums_s[...] = jnp.zeros_like(sums_s)
        cnts_s[...] = jnp.zeros_like(cnts_s)

    sums_s[...] += bsum
    cnts_s[...] += bcnt

    @pl.when(i == NBLK - 1)
    def _():
        out_ref[...] = sums_s[...] / jnp.maximum(cnts_s[...], 1.0) + lb_ref[...]


def _pool_tc(a0, a1, y2, d, batch_col, b2r, lw, lbr):
    return pl.pallas_call(
        _pool_body,
        grid=(NBLK,),
        in_specs=[
            pl.BlockSpec((RB, 1), lambda i: (i, 0)),
            pl.BlockSpec((RB, 1), lambda i: (i, 0)),
            pl.BlockSpec((RB, 1), lambda i: (i, 0)),
            pl.BlockSpec((RB, 1), lambda i: (i, 0)),
            pl.BlockSpec((RB, 1), lambda i: (i, 0)),
            pl.BlockSpec((1, D), lambda i: (0, 0)),
            pl.BlockSpec((D, 1), lambda i: (0, 0)),
            pl.BlockSpec((1, 1), lambda i: (0, 0)),
        ],
        out_specs=pl.BlockSpec((G, 1), lambda i: (0, 0)),
        out_shape=jax.ShapeDtypeStruct((G, 1), jnp.float32),
        scratch_shapes=[
            pltpu.VMEM((G, 1), jnp.float32),
            pltpu.VMEM((G, 1), jnp.float32),
        ],
    )(a0, a1, y2, d, batch_col, b2r, lw, lbr)


def kernel(x, edge_index, batch, W1, b1, W2, b2, lin_W, lin_b):
    src = edge_index[0].astype(jnp.int32)
    dst = edge_index[1].astype(jnp.int32)
    pad = E_PAD - E
    src_pad = jnp.arange(pad, dtype=jnp.int32) % N
    src_f = jnp.concatenate([src, src_pad])
    dst_pad = N + (jnp.arange(pad, dtype=jnp.int32) % (N_ACC - N - 1))
    dst_f = jnp.concatenate([dst, dst_pad])
    src_p = src_f.reshape(E_PAD // C, 1, C)
    dst_p = dst_f.reshape(E_PAD // C, 1, C)

    ones_c = jnp.ones((C,), jnp.float32)
    zeros_1d = jnp.zeros((ROWS_T,), jnp.float32)
    zeros_2d = jnp.zeros((ROWS_T, D), jnp.float32)

    _deg_sc, _agg_sc, _sagg_sc = _sc_kernels()

    degp = _deg_sc(dst_p, ones_c, zeros_1d)
    dp0 = degp[0, :N, None]
    dp1 = degp[1, :N, None]

    y, d = _yd_tc(x, W1, dp0, dp1)

    accp = _agg_sc(y, src_f, dst_f, zeros_2d)

    y2 = _h_tc(accp[0, :N], accp[1, :N], y, d,
               b1.reshape(1, D), W2, lin_W)

    y2_p = jnp.zeros((N_ACC,), jnp.float32).at[:N].set(y2.reshape(N))
    acc2p = _sagg_sc(y2_p, src_p, dst_p, zeros_1d)

    out = _pool_tc(acc2p[0, :N, None], acc2p[1, :N, None], y2, d,
                   batch.astype(jnp.int32).reshape(N, 1),
                   b2.reshape(1, D), lin_W, lin_b.reshape(1, 1))
    return out.reshape(G)


# speedup vs baseline: 38.8333x; 1.1189x over previous
import functools

import jax
import jax.numpy as jnp
from jax import lax
from jax.experimental import pallas as pl
from jax.experimental.pallas import tpu as pltpu
from jax.experimental.pallas import tpu_sc as plsc

N = 10000
E = 320000
D = 128
G = 256
NC = 2
NS = 16
NW = NC * NS
C = 128
CHUNKS_W = 80
E_PAD = NW * CHUNKS_W * C
EDGES_W = CHUNKS_W * C
N_ACC = NW * 320
ROWS_T = N_ACC // NS
CHUNKS_T = E_PAD // (NS * C)
EDGES_T = CHUNKS_T * C
DH = D // 2
RB = 400
NBLK = N // RB

def _worker_id():
    return lax.axis_index("s") * NC + lax.axis_index("c")


def _deg_body(dst2_hbm, ones_hbm, zeros_hbm, out_hbm, di_v, ones_v, sem, deg_sp):
    cid = lax.axis_index("c")
    sid = lax.axis_index("s")
    wid = _worker_id()
    pltpu.sync_copy(zeros_hbm, deg_sp.at[pl.ds(sid * ROWS_T, ROWS_T)])
    pltpu.sync_copy(ones_hbm, ones_v)
    pltpu.sync_copy(dst2_hbm.at[pl.ds(wid * CHUNKS_W, CHUNKS_W)], di_v)
    plsc.subcore_barrier()

    def fire(j, carry):
        pltpu.async_copy(ones_v, deg_sp.at[di_v.at[j, 0]], sem, add=True)
        return carry

    def drain(j, carry):
        pltpu.make_async_copy(ones_v, deg_sp.at[di_v.at[0, 0]], sem).wait()
        return carry

    lax.fori_loop(0, CHUNKS_W, fire, 0, unroll=False)
    lax.fori_loop(0, CHUNKS_W, drain, 0, unroll=False)
    plsc.subcore_barrier()
    pltpu.sync_copy(
        deg_sp.at[pl.ds(sid * ROWS_T, ROWS_T)],
        out_hbm.at[cid, pl.ds(sid * ROWS_T, ROWS_T)],
    )


def _agg_body(y_hbm, src2_hbm, dst2_hbm, zeros_hbm, out_hbm,
              si0, si1, di0, di1, r0, r1,
              g0, g1, s0, s1, i0, i1, d0, d1, acc_sp):
    cid = lax.axis_index("c")
    sid = lax.axis_index("s")
    wid = _worker_id()
    pltpu.sync_copy(zeros_hbm, acc_sp.at[pl.ds(sid * ROWS_T, ROWS_T)])
    plsc.subcore_barrier()

    base = wid * EDGES_W
    last = CHUNKS_W - 1

    pltpu.async_copy(src2_hbm.at[pl.ds(base, C)], si0, i0)
    pltpu.async_copy(dst2_hbm.at[pl.ds(base, C)], di0, d0)
    pltpu.async_copy(src2_hbm.at[pl.ds(base + C, C)], si1, i1)
    pltpu.async_copy(dst2_hbm.at[pl.ds(base + C, C)], di1, d1)
    pltpu.make_async_copy(src2_hbm.at[pl.ds(base, C)], si0, i0).wait()
    pltpu.async_copy(y_hbm.at[si0], r0, g0)
    pltpu.make_async_copy(src2_hbm.at[pl.ds(base, C)], si1, i1).wait()
    pltpu.async_copy(y_hbm.at[si1], r1, g1)

    def body(i, carry):
        j0 = 2 * i
        j1 = 2 * i + 1
        pltpu.make_async_copy(y_hbm.at[si0], r0, g0).wait()
        pltpu.make_async_copy(dst2_hbm.at[pl.ds(base, C)], di0, d0).wait()
        pltpu.async_copy(r0, acc_sp.at[di0], s0, add=True)
        pltpu.make_async_copy(y_hbm.at[si1], r1, g1).wait()
        pltpu.make_async_copy(dst2_hbm.at[pl.ds(base, C)], di1, d1).wait()
        pltpu.async_copy(r1, acc_sp.at[di1], s1, add=True)
        pltpu.make_async_copy(r0, acc_sp.at[di0], s0).wait()

        @pl.when(j0 + 2 <= last)
        def _():
            off = base + (j0 + 2) * C
            pltpu.async_copy(src2_hbm.at[pl.ds(off, C)], si0, i0)
            pltpu.async_copy(dst2_hbm.at[pl.ds(off, C)], di0, d0)
            pltpu.make_async_copy(src2_hbm.at[pl.ds(base, C)], si0, i0).wait()
            pltpu.async_copy(y_hbm.at[si0], r0, g0)

        pltpu.make_async_copy(r1, acc_sp.at[di1], s1).wait()

        @pl.when(j1 + 2 <= last)
        def _():
            off = base + (j1 + 2) * C
            pltpu.async_copy(src2_hbm.at[pl.ds(off, C)], si1, i1)
            pltpu.async_copy(dst2_hbm.at[pl.ds(off, C)], di1, d1)
            pltpu.make_async_copy(src2_hbm.at[pl.ds(base, C)], si1, i1).wait()
            pltpu.async_copy(y_hbm.at[si1], r1, g1)

        return carry

    lax.fori_loop(0, CHUNKS_W // 2, body, 0, unroll=False)
    plsc.subcore_barrier()
    pltpu.sync_copy(
        acc_sp.at[pl.ds(sid * ROWS_T, ROWS_T)],
        out_hbm.at[cid, pl.ds(sid * ROWS_T, ROWS_T)],
    )


def _sagg_body(y2_hbm, src2_hbm, dst2_hbm, zeros_hbm, out_hbm,
               si_v, di_v, vals_v, y2_t, gsem, ssem, acc_sp):
    cid = lax.axis_index("c")
    sid = lax.axis_index("s")
    wid = _worker_id()
    pltpu.sync_copy(zeros_hbm, acc_sp.at[pl.ds(sid * ROWS_T, ROWS_T)])
    pltpu.sync_copy(y2_hbm, y2_t)
    pltpu.sync_copy(src2_hbm.at[pl.ds(wid * CHUNKS_W, CHUNKS_W)], si_v)
    pltpu.sync_copy(dst2_hbm.at[pl.ds(wid * CHUNKS_W, CHUNKS_W)], di_v)
    plsc.subcore_barrier()

    def gather_fire(j, carry):
        for k in range(C // 16):
            idx16 = si_v[j, 0, pl.ds(k * 16, 16)]
            vals_v[j, 0, pl.ds(k * 16, 16)] = plsc.load_gather(y2_t, [idx16])
        pltpu.async_copy(vals_v.at[j, 0], acc_sp.at[di_v.at[j, 0]], ssem, add=True)
        return carry

    def drain_scatter(j, carry):
        pltpu.make_async_copy(vals_v.at[0, 0], acc_sp.at[di_v.at[0, 0]], ssem).wait()
        return carry

    lax.fori_loop(0, CHUNKS_W, gather_fire, 0, unroll=False)
    lax.fori_loop(0, CHUNKS_W, drain_scatter, 0, unroll=False)
    plsc.subcore_barrier()
    pltpu.sync_copy(
        acc_sp.at[pl.ds(sid * ROWS_T, ROWS_T)],
        out_hbm.at[cid, pl.ds(sid * ROWS_T, ROWS_T)],
    )


@functools.cache
def _sc_kernels():
    mesh = plsc.VectorSubcoreMesh(
        core_axis_name="c", subcore_axis_name="s", num_cores=NC, num_subcores=NS
    )
    deg = pl.kernel(
        _deg_body,
        out_type=jax.ShapeDtypeStruct((NC, N_ACC), jnp.float32),
        mesh=mesh,
        scratch_types=[
            pltpu.VMEM((CHUNKS_W, 1, C), jnp.int32),
            pltpu.VMEM((C,), jnp.float32),
            pltpu.SemaphoreType.DMA,
            pltpu.VMEM_SHARED((N_ACC,), jnp.float32),
        ],
    )
    agg = pl.kernel(
        _agg_body,
        out_type=jax.ShapeDtypeStruct((NC, N_ACC, D), jnp.float32),
        mesh=mesh,
        scratch_types=[
            pltpu.VMEM((C,), jnp.int32),
            pltpu.VMEM((C,), jnp.int32),
            pltpu.VMEM((C,), jnp.int32),
            pltpu.VMEM((C,), jnp.int32),
            pltpu.VMEM((C, D), jnp.float32),
            pltpu.VMEM((C, D), jnp.float32),
            pltpu.SemaphoreType.DMA,
            pltpu.SemaphoreType.DMA,
            pltpu.SemaphoreType.DMA,
            pltpu.SemaphoreType.DMA,
            pltpu.SemaphoreType.DMA,
            pltpu.SemaphoreType.DMA,
            pltpu.SemaphoreType.DMA,
            pltpu.SemaphoreType.DMA,
            pltpu.VMEM_SHARED((N_ACC, D), jnp.float32),
        ],
    )
    sagg = pl.kernel(
        _sagg_body,
        out_type=jax.ShapeDtypeStruct((NC, N_ACC), jnp.float32),
        mesh=mesh,
        compiler_params=pltpu.CompilerParams(needs_layout_passes=False),
        scratch_types=[
            pltpu.VMEM((CHUNKS_W, 1, C), jnp.int32),
            pltpu.VMEM((CHUNKS_W, 1, C), jnp.int32),
            pltpu.VMEM((CHUNKS_W, 1, C), jnp.float32),
            pltpu.VMEM((N_ACC,), jnp.float32),
            pltpu.SemaphoreType.DMA,
            pltpu.SemaphoreType.DMA,
            pltpu.VMEM_SHARED((N_ACC,), jnp.float32),
        ],
    )
    return deg, agg, sagg


def _yd_body(x_ref, w1_ref, dp0_ref, dp1_ref, y_ref, d_ref):
    d = lax.rsqrt(1.0 + dp0_ref[...] + dp1_ref[...])
    xw = jnp.dot(x_ref[...], w1_ref[...], preferred_element_type=jnp.float32)
    y_ref[...] = xw * d
    d_ref[...] = d


def _yd_tc(x, w1, dp0, dp1):
    return pl.pallas_call(
        _yd_body,
        out_shape=[
            jax.ShapeDtypeStruct((N, D), jnp.float32),
            jax.ShapeDtypeStruct((N, 1), jnp.float32),
        ],
    )(x, w1, dp0, dp1)


def _h_body(a0_ref, a1_ref, y_ref, d_ref, b1_ref, w2_ref, lw_ref, y2_ref):
    d = d_ref[...]
    agg = a0_ref[...] + a1_ref[...]
    pre = d * (agg + y_ref[...]) + b1_ref[...]
    h1 = jnp.maximum(pre, 0.0)
    wv = jnp.dot(w2_ref[...], lw_ref[...], preferred_element_type=jnp.float32)
    s = jnp.dot(h1, wv, preferred_element_type=jnp.float32)
    y2_ref[...] = d * s


def _h_tc(a0, a1, y, d, b1r, w2, lw):
    return pl.pallas_call(
        _h_body,
        out_shape=jax.ShapeDtypeStruct((N, 1), jnp.float32),
    )(a0, a1, y, d, b1r, w2, lw)


def _pool_body(a0_ref, a1_ref, y2_ref, d_ref, batch_ref, b2_ref, lw_ref,
               lb_ref, out_ref):
    d = d_ref[...]
    c2 = jnp.dot(b2_ref[...], lw_ref[...], preferred_element_type=jnp.float32)
    t = d * (a0_ref[...] + a1_ref[...]) + d * y2_ref[...] + c2
    gid = lax.broadcasted_iota(jnp.int32, (N, G), 1)
    onehot = (batch_ref[...] == gid).astype(jnp.float32)
    bsum = lax.dot_general(onehot, t, (((0,), (0,)), ((), ())),
                           preferred_element_type=jnp.float32)
    ones = jnp.ones((N, 1), jnp.float32)
    bcnt = lax.dot_general(onehot, ones, (((0,), (0,)), ((), ())),
                           preferred_element_type=jnp.float32)
    out_ref[...] = bsum / jnp.maximum(bcnt, 1.0) + lb_ref[...]


def _pool_tc(a0, a1, y2, d, batch_col, b2r, lw, lbr):
    return pl.pallas_call(
        _pool_body,
        out_shape=jax.ShapeDtypeStruct((G, 1), jnp.float32),
    )(a0, a1, y2, d, batch_col, b2r, lw, lbr)


def kernel(x, edge_index, batch, W1, b1, W2, b2, lin_W, lin_b):
    src = edge_index[0].astype(jnp.int32)
    dst = edge_index[1].astype(jnp.int32)
    pad = E_PAD - E
    src_pad = jnp.arange(pad, dtype=jnp.int32) % N
    src_f = jnp.concatenate([src, src_pad])
    dst_pad = N + (jnp.arange(pad, dtype=jnp.int32) % (N_ACC - N - 1))
    dst_f = jnp.concatenate([dst, dst_pad])
    src_p = src_f.reshape(E_PAD // C, 1, C)
    dst_p = dst_f.reshape(E_PAD // C, 1, C)

    ones_c = jnp.ones((C,), jnp.float32)
    zeros_1d = jnp.zeros((ROWS_T,), jnp.float32)
    zeros_2d = jnp.zeros((ROWS_T, D), jnp.float32)

    _deg_sc, _agg_sc, _sagg_sc = _sc_kernels()

    degp = _deg_sc(dst_p, ones_c, zeros_1d)
    dp0 = degp[0, :N, None]
    dp1 = degp[1, :N, None]

    y, d = _yd_tc(x, W1, dp0, dp1)

    accp = _agg_sc(y, src_f, dst_f, zeros_2d)

    y2 = _h_tc(accp[0, :N], accp[1, :N], y, d,
               b1.reshape(1, D), W2, lin_W)

    y2_p = jnp.zeros((N_ACC,), jnp.float32).at[:N].set(y2.reshape(N))
    acc2p = _sagg_sc(y2_p, src_p, dst_p, zeros_1d)

    out = _pool_tc(acc2p[0, :N, None], acc2p[1, :N, None], y2, d,
                   batch.astype(jnp.int32).reshape(N, 1),
                   b2.reshape(1, D), lin_W, lin_b.reshape(1, 1))
    return out.reshape(G)


# speedup vs baseline: 39.4963x; 1.0171x over previous
import functools

import jax
import jax.numpy as jnp
from jax import lax
from jax.experimental import pallas as pl
from jax.experimental.pallas import tpu as pltpu
from jax.experimental.pallas import tpu_sc as plsc

N = 10000
E = 320000
D = 128
G = 256
NC = 2
NS = 16
NW = NC * NS
C = 128
CHUNKS_W = 80
E_PAD = NW * CHUNKS_W * C
CA = 120
CHUNKS_A = 84
E_PAD_A = NW * CHUNKS_A * CA
EDGES_A = CHUNKS_A * CA
N_ACC = NW * 320
ROWS_T = N_ACC // NS

def _worker_id():
    return lax.axis_index("s") * NC + lax.axis_index("c")


def _deg_body(dst2_hbm, ones_hbm, zeros_hbm, out_hbm, di_v, ones_v, sem, deg_sp):
    cid = lax.axis_index("c")
    sid = lax.axis_index("s")
    wid = _worker_id()
    pltpu.sync_copy(zeros_hbm, deg_sp.at[pl.ds(sid * ROWS_T, ROWS_T)])
    pltpu.sync_copy(ones_hbm, ones_v)
    pltpu.sync_copy(dst2_hbm.at[pl.ds(wid * CHUNKS_W, CHUNKS_W)], di_v)
    plsc.subcore_barrier()

    def fire(j, carry):
        pltpu.async_copy(ones_v, deg_sp.at[di_v.at[j, 0]], sem, add=True)
        return carry

    def drain(j, carry):
        pltpu.make_async_copy(ones_v, deg_sp.at[di_v.at[0, 0]], sem).wait()
        return carry

    lax.fori_loop(0, CHUNKS_W, fire, 0, unroll=False)
    lax.fori_loop(0, CHUNKS_W, drain, 0, unroll=False)
    plsc.subcore_barrier()
    pltpu.sync_copy(
        deg_sp.at[pl.ds(sid * ROWS_T, ROWS_T)],
        out_hbm.at[cid, pl.ds(sid * ROWS_T, ROWS_T)],
    )


def _agg_body(y_hbm, src2_hbm, dst2_hbm, zeros_hbm, out_hbm,
              siA, siB, siC, diA, diB, diC, rA, rB, rC,
              gA, gB, gC, sA, sB, sC, iA, iB, iC, dA, dB, dC, acc_sp):
    cid = lax.axis_index("c")
    sid = lax.axis_index("s")
    wid = _worker_id()
    pltpu.sync_copy(zeros_hbm, acc_sp.at[pl.ds(sid * ROWS_T, ROWS_T)])
    plsc.subcore_barrier()

    base = wid * EDGES_A
    last = CHUNKS_A - 1

    pltpu.async_copy(src2_hbm.at[pl.ds(base, CA)], siA, iA)
    pltpu.async_copy(dst2_hbm.at[pl.ds(base, CA)], diA, dA)
    pltpu.async_copy(src2_hbm.at[pl.ds(base + CA, CA)], siB, iB)
    pltpu.async_copy(dst2_hbm.at[pl.ds(base + CA, CA)], diB, dB)
    pltpu.async_copy(src2_hbm.at[pl.ds(base + 2 * CA, CA)], siC, iC)
    pltpu.async_copy(dst2_hbm.at[pl.ds(base + 2 * CA, CA)], diC, dC)
    pltpu.make_async_copy(src2_hbm.at[pl.ds(base, CA)], siA, iA).wait()
    pltpu.async_copy(y_hbm.at[siA], rA, gA)
    pltpu.make_async_copy(src2_hbm.at[pl.ds(base, CA)], siB, iB).wait()
    pltpu.async_copy(y_hbm.at[siB], rB, gB)
    pltpu.make_async_copy(src2_hbm.at[pl.ds(base, CA)], siC, iC).wait()
    pltpu.async_copy(y_hbm.at[siC], rC, gC)

    lanes = ((siA, diA, rA, gA, sA, iA, dA, 0),
             (siB, diB, rB, gB, sB, iB, dB, 1),
             (siC, diC, rC, gC, sC, iC, dC, 2))

    def body(m, carry):
        j = 3 * m
        for (si, di, r, g, s, ii, dd, o) in lanes:
            pltpu.make_async_copy(y_hbm.at[si], r, g).wait()
            pltpu.make_async_copy(dst2_hbm.at[pl.ds(base, CA)], di, dd).wait()
            pltpu.async_copy(r, acc_sp.at[di], s, add=True)
            pltpu.make_async_copy(r, acc_sp.at[di], s).wait()
        for (si, di, r, g, s, ii, dd, o) in lanes:

            @pl.when(j + o + 3 <= last)
            def _(si=si, di=di, r=r, g=g, ii=ii, dd=dd, o=o):
                off = base + (j + o + 3) * CA
                pltpu.async_copy(src2_hbm.at[pl.ds(off, CA)], si, ii)
                pltpu.async_copy(dst2_hbm.at[pl.ds(off, CA)], di, dd)
                pltpu.make_async_copy(src2_hbm.at[pl.ds(base, CA)], si, ii).wait()
                pltpu.async_copy(y_hbm.at[si], r, g)

        return carry

    lax.fori_loop(0, CHUNKS_A // 3, body, 0, unroll=False)
    plsc.subcore_barrier()
    pltpu.sync_copy(
        acc_sp.at[pl.ds(sid * ROWS_T, ROWS_T)],
        out_hbm.at[cid, pl.ds(sid * ROWS_T, ROWS_T)],
    )


def _sagg_body(y2_hbm, src2_hbm, dst2_hbm, zeros_hbm, out_hbm,
               si_v, di_v, vals_v, y2_t, gsem, ssem, acc_sp):
    cid = lax.axis_index("c")
    sid = lax.axis_index("s")
    wid = _worker_id()
    pltpu.sync_copy(zeros_hbm, acc_sp.at[pl.ds(sid * ROWS_T, ROWS_T)])
    pltpu.sync_copy(y2_hbm, y2_t)
    pltpu.sync_copy(src2_hbm.at[pl.ds(wid * CHUNKS_W, CHUNKS_W)], si_v)
    pltpu.sync_copy(dst2_hbm.at[pl.ds(wid * CHUNKS_W, CHUNKS_W)], di_v)
    plsc.subcore_barrier()

    def gather_fire(j, carry):
        for k in range(C // 16):
            idx16 = si_v[j, 0, pl.ds(k * 16, 16)]
            vals_v[j, 0, pl.ds(k * 16, 16)] = plsc.load_gather(y2_t, [idx16])
        pltpu.async_copy(vals_v.at[j, 0], acc_sp.at[di_v.at[j, 0]], ssem, add=True)
        return carry

    def drain_scatter(j, carry):
        pltpu.make_async_copy(vals_v.at[0, 0], acc_sp.at[di_v.at[0, 0]], ssem).wait()
        return carry

    lax.fori_loop(0, CHUNKS_W, gather_fire, 0, unroll=False)
    lax.fori_loop(0, CHUNKS_W, drain_scatter, 0, unroll=False)
    plsc.subcore_barrier()
    pltpu.sync_copy(
        acc_sp.at[pl.ds(sid * ROWS_T, ROWS_T)],
        out_hbm.at[cid, pl.ds(sid * ROWS_T, ROWS_T)],
    )


@functools.cache
def _sc_kernels():
    mesh = plsc.VectorSubcoreMesh(
        core_axis_name="c", subcore_axis_name="s", num_cores=NC, num_subcores=NS
    )
    deg = pl.kernel(
        _deg_body,
        out_type=jax.ShapeDtypeStruct((NC, N_ACC), jnp.float32),
        mesh=mesh,
        scratch_types=[
            pltpu.VMEM((CHUNKS_W, 1, C), jnp.int32),
            pltpu.VMEM((C,), jnp.float32),
            pltpu.SemaphoreType.DMA,
            pltpu.VMEM_SHARED((N_ACC,), jnp.float32),
        ],
    )
    agg = pl.kernel(
        _agg_body,
        out_type=jax.ShapeDtypeStruct((NC, N_ACC, D), jnp.float32),
        mesh=mesh,
        scratch_types=(
            [pltpu.VMEM((CA,), jnp.int32)] * 6
            + [pltpu.VMEM((CA, D), jnp.float32)] * 3
            + [pltpu.SemaphoreType.DMA] * 12
            + [pltpu.VMEM_SHARED((N_ACC, D), jnp.float32)]
        ),
    )
    sagg = pl.kernel(
        _sagg_body,
        out_type=jax.ShapeDtypeStruct((NC, N_ACC), jnp.float32),
        mesh=mesh,
        compiler_params=pltpu.CompilerParams(needs_layout_passes=False),
        scratch_types=[
            pltpu.VMEM((CHUNKS_W, 1, C), jnp.int32),
            pltpu.VMEM((CHUNKS_W, 1, C), jnp.int32),
            pltpu.VMEM((CHUNKS_W, 1, C), jnp.float32),
            pltpu.VMEM((N_ACC,), jnp.float32),
            pltpu.SemaphoreType.DMA,
            pltpu.SemaphoreType.DMA,
            pltpu.VMEM_SHARED((N_ACC,), jnp.float32),
        ],
    )
    return deg, agg, sagg


def _yd_body(x_ref, w1_ref, dp0_ref, dp1_ref, y_ref, d_ref):
    d = lax.rsqrt(1.0 + dp0_ref[...] + dp1_ref[...])
    xw = jnp.dot(x_ref[...], w1_ref[...], preferred_element_type=jnp.float32)
    y_ref[...] = xw * d
    d_ref[...] = d


def _yd_tc(x, w1, dp0, dp1):
    return pl.pallas_call(
        _yd_body,
        out_shape=[
            jax.ShapeDtypeStruct((N, D), jnp.float32),
            jax.ShapeDtypeStruct((N, 1), jnp.float32),
        ],
    )(x, w1, dp0, dp1)


def _h_body(a0_ref, a1_ref, y_ref, d_ref, b1_ref, w2_ref, lw_ref, y2_ref):
    d = d_ref[...]
    agg = a0_ref[...] + a1_ref[...]
    pre = d * (agg + y_ref[...]) + b1_ref[...]
    h1 = jnp.maximum(pre, 0.0)
    wv = jnp.dot(w2_ref[...], lw_ref[...], preferred_element_type=jnp.float32)
    s = jnp.dot(h1, wv, preferred_element_type=jnp.float32)
    y2_ref[...] = d * s


def _h_tc(a0, a1, y, d, b1r, w2, lw):
    return pl.pallas_call(
        _h_body,
        out_shape=jax.ShapeDtypeStruct((N, 1), jnp.float32),
    )(a0, a1, y, d, b1r, w2, lw)


def _pool_body(a0_ref, a1_ref, y2_ref, d_ref, batch_ref, b2_ref, lw_ref,
               lb_ref, out_ref):
    d = d_ref[...]
    c2 = jnp.dot(b2_ref[...], lw_ref[...], preferred_element_type=jnp.float32)
    t = d * (a0_ref[...] + a1_ref[...]) + d * y2_ref[...] + c2
    gid = lax.broadcasted_iota(jnp.int32, (N, G), 1)
    onehot = (batch_ref[...] == gid).astype(jnp.float32)
    bsum = lax.dot_general(onehot, t, (((0,), (0,)), ((), ())),
                           preferred_element_type=jnp.float32)
    ones = jnp.ones((N, 1), jnp.float32)
    bcnt = lax.dot_general(onehot, ones, (((0,), (0,)), ((), ())),
                           preferred_element_type=jnp.float32)
    out_ref[...] = bsum / jnp.maximum(bcnt, 1.0) + lb_ref[...]


def _pool_tc(a0, a1, y2, d, batch_col, b2r, lw, lbr):
    return pl.pallas_call(
        _pool_body,
        out_shape=jax.ShapeDtypeStruct((G, 1), jnp.float32),
    )(a0, a1, y2, d, batch_col, b2r, lw, lbr)


def kernel(x, edge_index, batch, W1, b1, W2, b2, lin_W, lin_b):
    src = edge_index[0].astype(jnp.int32)
    dst = edge_index[1].astype(jnp.int32)
    pad_a = E_PAD_A - E
    src_f = jnp.concatenate([src, jnp.arange(pad_a, dtype=jnp.int32) % N])
    dst_f = jnp.concatenate(
        [dst, N + (jnp.arange(pad_a, dtype=jnp.int32) % (N_ACC - N - 1))])
    pad = E_PAD - E
    src_p = jnp.concatenate(
        [src, jnp.arange(pad, dtype=jnp.int32) % N]).reshape(E_PAD // C, 1, C)
    dst_p = jnp.concatenate(
        [dst, N + (jnp.arange(pad, dtype=jnp.int32) % (N_ACC - N - 1))]
    ).reshape(E_PAD // C, 1, C)

    ones_c = jnp.ones((C,), jnp.float32)
    zeros_1d = jnp.zeros((ROWS_T,), jnp.float32)
    zeros_2d = jnp.zeros((ROWS_T, D), jnp.float32)

    _deg_sc, _agg_sc, _sagg_sc = _sc_kernels()

    degp = _deg_sc(dst_p, ones_c, zeros_1d)
    dp0 = degp[0, :N, None]
    dp1 = degp[1, :N, None]

    y, d = _yd_tc(x, W1, dp0, dp1)

    accp = _agg_sc(y, src_f, dst_f, zeros_2d)

    y2 = _h_tc(accp[0, :N], accp[1, :N], y, d,
               b1.reshape(1, D), W2, lin_W)

    y2_p = jnp.zeros((N_ACC,), jnp.float32).at[:N].set(y2.reshape(N))
    acc2p = _sagg_sc(y2_p, src_p, dst_p, zeros_1d)

    out = _pool_tc(acc2p[0, :N, None], acc2p[1, :N, None], y2, d,
                   batch.astype(jnp.int32).reshape(N, 1),
                   b2.reshape(1, D), lin_W, lin_b.reshape(1, 1))
    return out.reshape(G)
